# baseline (speedup 1.0000x reference)
import jax
import jax.numpy as jnp
from jax import lax
from jax.experimental import pallas as pl
from jax.experimental.pallas import tpu as pltpu
from jax.experimental.pallas import tpu_sc as plsc

D_MODEL = 64
SCALE = 8.0

NC = 2
NS = 16
NW = NC * NS

A = 4096
T = 200
AB = A // NW
NBUF = 4


def _body(x_hbm, table_hbm, out_hbm, xblk, xt, rows, obuf, gsem, dsem):
    wid = lax.axis_index("s") * NC + lax.axis_index("c")
    iota = lax.iota(jnp.int32, 16)

    pltpu.sync_copy(x_hbm.at[pl.ds(wid * AB, AB)], xblk)

    def trans_t(t, carry):
        tvec = jnp.full((16,), 0, jnp.int32) + t
        for g in range(AB // 16):
            v = plsc.load_gather(xblk, [iota + (g * 16), tvec])
            xt[t, pl.ds(g * 16, 16)] = v
        return carry

    lax.fori_loop(0, T, trans_t, 0)

    def fire_gather(t, b):
        pltpu.async_copy(table_hbm.at[xt.at[t]], rows.at[b], gsem.at[b])

    def wait_gather(b):
        pltpu.make_async_copy(
            table_hbm.at[pl.ds(0, AB)], rows.at[b], gsem.at[b]
        ).wait()

    def fire_drain(t, b):
        pltpu.async_copy(obuf.at[b], out_hbm.at[t, :, wid], dsem.at[b])

    def wait_drain(b):
        pltpu.make_async_copy(
            obuf.at[b], out_hbm.at[0, :, 0], dsem.at[b]
        ).wait()

    for b in range(NBUF - 1):
        fire_gather(b, b)

    def outer(g, carry):
        for b in range(NBUF):
            t = g * NBUF + b
            bn = (b + NBUF - 1) % NBUF

            @pl.when(t >= 1)
            def _():
                wait_drain(bn)

            @pl.when(t + NBUF - 1 < T)
            def _():
                fire_gather(t + NBUF - 1, bn)

            wait_gather(b)

            def build(d, carry2):
                dq = d // 8
                dr = d % 8
                dvec = jnp.full((16,), 0, jnp.int32) + d
                for ag in range(AB // 16):
                    v = plsc.load_gather(rows.at[b], [iota + (ag * 16), dvec])
                    obuf[b, dq, dr, pl.ds(ag * 16, 16)] = v * SCALE
                return carry2

            lax.fori_loop(0, D_MODEL, build, 0)

            fire_drain(t, b)
        return carry

    lax.fori_loop(0, T // NBUF, outer, 0)
    wait_drain((T - 1) % NBUF)


@jax.jit
def _run(x, table):
    mesh = plsc.VectorSubcoreMesh(core_axis_name="c", subcore_axis_name="s")
    f = pl.kernel(
        _body,
        mesh=mesh,
        out_type=jax.ShapeDtypeStruct((T, 8, NW, 8, AB), jnp.float32),
        scratch_types=[
            pltpu.VMEM((AB, T), jnp.int32),
            pltpu.VMEM((T, AB), jnp.int32),
            pltpu.VMEM((NBUF, AB, D_MODEL), jnp.float32),
            pltpu.VMEM((NBUF, 8, 8, AB), jnp.float32),
            pltpu.SemaphoreType.DMA((NBUF,)),
            pltpu.SemaphoreType.DMA((NBUF,)),
        ],
        compiler_params=pltpu.CompilerParams(
            use_tc_tiling_on_sc=False, needs_layout_passes=False
        ),
    )
    return f(x, table)


def kernel(x, table):
    out5 = _run(x.astype(jnp.int32), table)
    return out5.transpose(2, 4, 0, 1, 3).reshape(A, T, D_MODEL)


# speedup vs baseline: 1.1704x; 1.1704x over previous
import jax
import jax.numpy as jnp
from jax import lax
from jax.experimental import pallas as pl
from jax.experimental.pallas import tpu as pltpu
from jax.experimental.pallas import tpu_sc as plsc

D_MODEL = 64
SCALE = 8.0

NC = 2
NS = 16
NW = NC * NS

A = 4096
T = 200
AB = A // NW
PER_W = AB * T
NBUF = 2
VP = 500000


def _body(x_hbm, table_hbm, out_hbm, pidx, par, rows, cbuf, gsem, dsem):
    wid = lax.axis_index("s") * NC + lax.axis_index("c")
    iota = lax.iota(jnp.int32, 16)
    tt = table_hbm

    pltpu.sync_copy(x_hbm.at[pl.ds(wid * PER_W, PER_W)], pidx)

    def prep(i, carry):
        for u in range(4):
            o = (i * 4 + u) * 16
            v = pidx[pl.ds(o, 16)]
            par[pl.ds(o, 16)] = (v & 1) * D_MODEL
            pidx[pl.ds(o, 16)] = lax.shift_right_logical(v, 1)
        return carry

    lax.fori_loop(0, PER_W // 64, prep, 0)

    def fire_gather(al, b):
        o = al * T
        pltpu.async_copy(tt.at[pidx.at[pl.ds(o, 128)]],
                         rows.at[b, pl.ds(0, 128)], gsem.at[b])
        pltpu.async_copy(tt.at[pidx.at[pl.ds(o + 128, T - 128)]],
                         rows.at[b, pl.ds(128, T - 128)], gsem.at[b])

    def wait_gather(b):
        pltpu.make_async_copy(tt.at[pl.ds(0, T)], rows.at[b], gsem.at[b]).wait()

    def fire_drain(al, b):
        aq = (wid * AB + al) // 8
        a8 = (wid * AB + al) % 8
        pltpu.async_copy(cbuf.at[b], out_hbm.at[aq, :, a8], dsem.at[b])

    def wait_drain(b):
        pltpu.make_async_copy(cbuf.at[b], out_hbm.at[0, :, 0], dsem.at[b]).wait()

    for b in range(NBUF - 1):
        fire_gather(b, b)

    def sel_one(b, pv, j, row, crow_base, crow_off):
        pj = pv[jnp.full((16,), j, jnp.int32)]
        rvec = jnp.full((16,), 0, jnp.int32) + row
        for s in range(D_MODEL // 16):
            v = plsc.load_gather(rows.at[b], [rvec, pj + (iota + s * 16)])
            q = crow_off + s
            cbuf[b, crow_base + q // 8, pl.ds((q % 8) * 16, 16)] = v * SCALE

    def outer(g, carry):
        for b in range(NBUF):
            al = g * NBUF + b
            bn = (b + NBUF - 1) % NBUF

            @pl.when(al + NBUF - 1 < AB)
            def _():
                fire_gather(al + NBUF - 1, bn)

            wait_gather(b)

            @pl.when(al >= NBUF)
            def _():
                wait_drain(b)

            def grp_loop(grp, carry2):
                pv = par[pl.ds(al * T + grp * 16, 16)]
                for j in range(16):
                    sel_one(b, pv, j, grp * 16 + j, grp * 8, 4 * j)
                return carry2

            lax.fori_loop(0, (T - 8) // 16, grp_loop, 0)

            pvt = par[pl.ds(al * T + (T - 16), 16)]
            for j in range(8, 16):
                row = (T - 16) + j
                q = row * 4
                sel_one(b, pvt, j, row, q // 8, q % 8)

            fire_drain(al, b)
        return carry

    lax.fori_loop(0, AB // NBUF, outer, 0)
    for b in range(NBUF):
        wait_drain(b)


@jax.jit
def _run(xf, table):
    mesh = plsc.VectorSubcoreMesh(core_axis_name="c", subcore_axis_name="s")
    f = pl.kernel(
        _body,
        mesh=mesh,
        out_type=jax.ShapeDtypeStruct((A // 8, T * D_MODEL // 128, 8, 128), jnp.float32),
        scratch_types=[
            pltpu.VMEM((PER_W,), jnp.int32),
            pltpu.VMEM((PER_W,), jnp.int32),
            pltpu.VMEM((NBUF, T, 2 * D_MODEL), jnp.float32),
            pltpu.VMEM((NBUF, T * D_MODEL // 128, 128), jnp.float32),
            pltpu.SemaphoreType.DMA((NBUF,)),
            pltpu.SemaphoreType.DMA((NBUF,)),
        ],
        compiler_params=pltpu.CompilerParams(
            use_tc_tiling_on_sc=False, needs_layout_passes=False
        ),
    )
    return f(xf, table)


def kernel(x, table):
    out4 = _run(x.reshape(A * T).astype(jnp.int32), table.reshape(VP, 2 * D_MODEL))
    return out4.transpose(0, 2, 1, 3).reshape(A, T, D_MODEL)


# speedup vs baseline: 1.4529x; 1.2414x over previous
import jax
import jax.numpy as jnp
from jax import lax
from jax.experimental import pallas as pl
from jax.experimental.pallas import tpu as pltpu
from jax.experimental.pallas import tpu_sc as plsc

D_MODEL = 64
SCALE = 8.0

NC = 2
NS = 16
NW = NC * NS

A = 4096
T = 200
AB = A // NW
PER_W = AB * T
NBUF = 2
VP = 500000


def _body(x_hbm, table_hbm, out_hbm, pidx, rows, cbuf, gsem, dsem):
    wid = lax.axis_index("s") * NC + lax.axis_index("c")
    iota = lax.iota(jnp.int32, 16)
    tt = table_hbm

    pltpu.sync_copy(x_hbm.at[pl.ds(wid * PER_W, PER_W)], pidx)

    def fire_gather(al, b):
        o = al * T
        pltpu.async_copy(tt.at[pidx.at[pl.ds(o, 128)]],
                         rows.at[b, pl.ds(0, 128)], gsem.at[b])
        pltpu.async_copy(tt.at[pidx.at[pl.ds(o + 128, T - 128)]],
                         rows.at[b, pl.ds(128, T - 128)], gsem.at[b])

    def wait_gather(b):
        pltpu.make_async_copy(tt.at[pl.ds(0, T)], rows.at[b], gsem.at[b]).wait()

    def fire_drain(al, b):
        aq = (wid * AB + al) // 8
        a8 = (wid * AB + al) % 8
        pltpu.async_copy(cbuf.at[b], out_hbm.at[aq, :, a8], dsem.at[b])

    def wait_drain(b):
        pltpu.make_async_copy(cbuf.at[b], out_hbm.at[0, :, 0], dsem.at[b]).wait()

    for b in range(NBUF - 1):
        fire_gather(b, b)

    def sel_one(b, j, row, crow_base, crow_off):
        for s in range(D_MODEL // 16):
            v = rows[b, row, pl.ds(s * 16, 16)]
            q = crow_off + s
            cbuf[b, crow_base + q // 8, pl.ds((q % 8) * 16, 16)] = v * SCALE

    def outer(g, carry):
        for b in range(NBUF):
            al = g * NBUF + b
            bn = (b + NBUF - 1) % NBUF

            @pl.when(al + NBUF - 1 < AB)
            def _():
                fire_gather(al + NBUF - 1, bn)

            wait_gather(b)

            @pl.when(al >= NBUF)
            def _():
                wait_drain(b)

            def grp_loop(grp, carry2):
                for j in range(8):
                    sel_one(b, j, grp * 8 + j, grp * 4, 4 * j)
                return carry2

            lax.fori_loop(0, T // 8, grp_loop, 0)

            fire_drain(al, b)
        return carry

    lax.fori_loop(0, AB // NBUF, outer, 0)
    for b in range(NBUF):
        wait_drain(b)


@jax.jit
def _run(xf, table):
    mesh = plsc.VectorSubcoreMesh(core_axis_name="c", subcore_axis_name="s")
    f = pl.kernel(
        _body,
        mesh=mesh,
        out_type=jax.ShapeDtypeStruct((A // 8, T * D_MODEL // 128, 8, 128), jnp.float32),
        scratch_types=[
            pltpu.VMEM((PER_W,), jnp.int32),
            pltpu.VMEM((NBUF, T, D_MODEL), jnp.float32),
            pltpu.VMEM((NBUF, T * D_MODEL // 128, 128), jnp.float32),
            pltpu.SemaphoreType.DMA((NBUF,)),
            pltpu.SemaphoreType.DMA((NBUF,)),
        ],
        compiler_params=pltpu.CompilerParams(
            use_tc_tiling_on_sc=False, needs_layout_passes=False
        ),
    )
    return f(xf, table)


def kernel(x, table):
    out4 = _run(x.reshape(A * T).astype(jnp.int32), table)
    return out4.transpose(0, 2, 1, 3).reshape(A, T, D_MODEL)
